# baseline (speedup 1.0000x reference)
import functools

import jax
import jax.numpy as jnp
from jax import lax
from jax.experimental import pallas as pl
from jax.experimental.pallas import tpu as pltpu
from jax.experimental.pallas import tpu_sc as plsc

_NUM_CORES = 2
_NUM_SUBCORES = 16
_NUM_WORKERS = _NUM_CORES * _NUM_SUBCORES
_LANES = 16


def _transpose_table(tT):
    D, V = tT.shape
    NFULL = V // 128
    TAILC = V - NFULL * 128
    mesh = plsc.VectorSubcoreMesh(core_axis_name="c", subcore_axis_name="s")

    @functools.partial(
        pl.kernel,
        out_type=jax.ShapeDtypeStruct((V // 2, 2 * D), jnp.float32),
        mesh=mesh,
        scratch_types=[
            pltpu.VMEM((2, D, 129), jnp.float32),
            pltpu.VMEM((2, 64, 128), jnp.float32),
            pltpu.SemaphoreType.DMA((2,)),
            pltpu.SemaphoreType.DMA((2,)),
        ],
        compiler_params=pltpu.CompilerParams(
            use_tc_tiling_on_sc=True, needs_layout_passes=False),
    )
    def trans(tT_hbm, tail_hbm, o2_hbm, vin, obuf, isem, osem):
        wid = lax.axis_index("s") * _NUM_CORES + lax.axis_index("c")
        n_my = (NFULL - wid + _NUM_WORKERS - 1) // _NUM_WORKERS

        qb = [lax.iota(jnp.int32, _LANES) + q * _LANES for q in range(D // _LANES)]

        def fire_in(c, width, sl):
            for g in range(D // 8):
                pltpu.async_copy(
                    tT_hbm.at[pl.ds(8 * g, 8), pl.ds(c * 128, width)],
                    vin.at[sl, pl.ds(8 * g, 8), pl.ds(0, width)],
                    isem.at[sl])

        def drain_in(width, sl):
            for g in range(D // 8):
                pltpu.make_async_copy(
                    tT_hbm.at[pl.ds(0, 8), pl.ds(0, width)],
                    vin.at[sl, pl.ds(0, 8), pl.ds(0, width)],
                    isem.at[sl]).wait()

        def drain_out(rows, sl):
            pltpu.make_async_copy(
                obuf.at[sl, pl.ds(0, rows)],
                o2_hbm.at[pl.ds(0, rows)], osem.at[sl]).wait()

        def transpose_block(rows, sl):
            src = vin.at[sl]
            dstb = obuf.at[sl]

            @plsc.parallel_loop(0, rows, step=1, unroll=8)
            def _t(l):
                lvec = jnp.full((_LANES,), l, jnp.int32)
                for q in range(D // _LANES):
                    vals = plsc.load_gather(src, [qb[q], lvec])
                    dstb[l // 2, pl.ds((l % 2) * D + q * _LANES, _LANES)] = vals

        @pl.when(wid == 0)
        def _tail():
            pltpu.sync_copy(tail_hbm, obuf.at[0, pl.ds(0, TAILC // 2)])
            pltpu.sync_copy(obuf.at[0, pl.ds(0, TAILC // 2)],
                            o2_hbm.at[pl.ds(NFULL * 64, TAILC // 2)])

        fire_in(wid, 128, 0)

        @pl.loop(0, 2 * ((n_my + 1) // 2), step=2)
        def _blk(j0):
            for sl in range(2):
                j = j0 + sl
                nsl = (sl + 1) % 2

                @pl.when(j < n_my)
                def _do():
                    c = wid + j * _NUM_WORKERS

                    @pl.when(j + 1 < n_my)
                    def _fire_next():
                        fire_in(wid + (j + 1) * _NUM_WORKERS, 128, nsl)

                    drain_in(128, sl)

                    @pl.when(j >= 2)
                    def _dprev():
                        drain_out(64, sl)

                    transpose_block(128, sl)
                    pltpu.async_copy(
                        obuf.at[sl], o2_hbm.at[pl.ds(c * 64, 64)], osem.at[sl])

        @pl.when(n_my >= 1)
        def _d0():
            drain_out(64, 0)

        @pl.when(n_my >= 2)
        def _d1():
            drain_out(64, 1)

    return trans


def _gather(xT3, table_lin, S, DG, BC, D):
    mesh = plsc.VectorSubcoreMesh(core_axis_name="c", subcore_axis_name="s")

    @functools.partial(
        pl.kernel,
        out_type=jax.ShapeDtypeStruct((S, DG, BC, 8, 128), jnp.float32),
        mesh=mesh,
        scratch_types=[
            pltpu.VMEM((S, 128), jnp.int32),
            pltpu.VMEM((2, 128, D), jnp.float32),
            pltpu.VMEM((2, D, 129), jnp.float32),
            pltpu.SemaphoreType.DMA((2,)),
            pltpu.SemaphoreType.DMA((2,)),
        ],
        compiler_params=pltpu.CompilerParams(
            use_tc_tiling_on_sc=False, needs_layout_passes=False),
    )
    def emb(idx_hbm, table_hbm, out_hbm, idx_v, vin, tbuf, gsem, osem):
        wid = lax.axis_index("s") * _NUM_CORES + lax.axis_index("c")
        pltpu.sync_copy(idx_hbm.at[:, wid], idx_v)

        base = [lax.iota(jnp.int32, _LANES) + c * _LANES for c in range(4)]

        def fire_gather(s, sl):
            pltpu.async_copy(table_hbm.at[idx_v.at[s]], vin.at[sl], gsem.at[sl])

        def drain_gather(sl):
            pltpu.make_async_copy(
                table_hbm.at[pl.ds(0, 128)], vin.at[sl], gsem.at[sl]).wait()

        def drain_out(sl):
            for dg in range(DG):
                pltpu.make_async_copy(
                    tbuf.at[sl, pl.ds(dg * 8, 8), pl.ds(0, 128)],
                    out_hbm.at[0, dg, wid], osem.at[sl]).wait()

        fire_gather(0, 0)

        @pl.loop(0, S, step=2)
        def _grp(s0):
            for sl in range(2):
                s = s0 + sl
                nsl = (sl + 1) % 2

                @pl.when(s + 1 < S)
                def _fire_next():
                    fire_gather(s + 1, nsl)

                drain_gather(sl)

                @pl.when(s >= 2)
                def _drain_prev():
                    drain_out(sl)

                dst = tbuf.at[sl]
                src = vin.at[sl]

                @plsc.parallel_loop(0, 128, step=1, unroll=8)
                def _transpose(l):
                    lvec = jnp.full((_LANES,), l, jnp.int32)
                    for c in range(4):
                        vals = src[l, pl.ds(c * _LANES, _LANES)]
                        plsc.store_scatter(dst, [base[c], lvec], vals)

                for dg in range(DG):
                    pltpu.async_copy(
                        tbuf.at[sl, pl.ds(dg * 8, 8), pl.ds(0, 128)],
                        out_hbm.at[s, dg, wid], osem.at[sl])

        for sl in range(2):
            drain_out(sl)

    return emb(xT3, table_lin)


def kernel(x, table):
    B, S = x.shape
    V, D = table.shape
    BC = B // 128
    DG = D // 8
    assert BC == _NUM_WORKERS and D == 64

    xT3 = x.T.reshape(S, BC, 128)
    NFULL = V // 128
    tail2 = table[NFULL * 128:, :].reshape((V - NFULL * 128) // 2, 2 * D)
    o2 = _transpose_table(table.T)(table.T, tail2)
    table_lin = o2.reshape(V, D)
    o5 = _gather(xT3, table_lin, S, DG, BC, D)
    return o5.transpose(2, 4, 0, 1, 3).reshape(B, S, D)


# speedup vs baseline: 1.2567x; 1.2567x over previous
import functools

import jax
import jax.numpy as jnp
from jax import lax
from jax.experimental import pallas as pl
from jax.experimental.pallas import tpu as pltpu
from jax.experimental.pallas import tpu_sc as plsc

_NUM_CORES = 2
_NUM_SUBCORES = 16
_NUM_WORKERS = _NUM_CORES * _NUM_SUBCORES
_LANES = 16


def kernel(x, table):
    B, S = x.shape
    V, D = table.shape
    BC = B // 128
    DG = D // 8
    assert BC == _NUM_WORKERS and D % 8 == 0

    xT3 = x.T.reshape(S, BC, 128)
    mesh = plsc.VectorSubcoreMesh(core_axis_name="c", subcore_axis_name="s")

    @functools.partial(
        pl.kernel,
        out_type=jax.ShapeDtypeStruct((S, DG, BC, 8, 128), jnp.float32),
        mesh=mesh,
        scratch_types=[
            pltpu.VMEM((S, 128), jnp.int32),
            pltpu.VMEM((2, 128, D), jnp.float32),
            pltpu.VMEM((2, D, 129), jnp.float32),
            pltpu.SemaphoreType.DMA((2,)),
            pltpu.SemaphoreType.DMA((2,)),
        ],
        compiler_params=pltpu.CompilerParams(
            use_tc_tiling_on_sc=False, needs_layout_passes=False),
    )
    def emb(idx_hbm, table_hbm, out_hbm, idx_v, vin, tbuf, gsem, osem):
        wid = lax.axis_index("s") * _NUM_CORES + lax.axis_index("c")
        pltpu.sync_copy(idx_hbm.at[:, wid], idx_v)

        base = [lax.iota(jnp.int32, _LANES) + c * _LANES for c in range(4)]

        def fire_gather(s, sl):
            pltpu.async_copy(table_hbm.at[idx_v.at[s]], vin.at[sl], gsem.at[sl])

        def drain_gather(sl):
            pltpu.make_async_copy(
                table_hbm.at[pl.ds(0, 128)], vin.at[sl], gsem.at[sl]).wait()

        def drain_out(sl):
            for dg in range(DG):
                pltpu.make_async_copy(
                    tbuf.at[sl, pl.ds(dg * 8, 8), pl.ds(0, 128)],
                    out_hbm.at[0, dg, wid], osem.at[sl]).wait()

        fire_gather(0, 0)

        @pl.loop(0, S, step=2)
        def _grp(s0):
            for sl in range(2):
                s = s0 + sl
                nsl = (sl + 1) % 2

                @pl.when(s + 1 < S)
                def _fire_next():
                    fire_gather(s + 1, nsl)

                drain_gather(sl)

                @pl.when(s >= 2)
                def _drain_prev():
                    drain_out(sl)

                dst = tbuf.at[sl]
                src = vin.at[sl]

                @plsc.parallel_loop(0, 128, step=1, unroll=8)
                def _transpose(l):
                    lvec = jnp.full((_LANES,), l, jnp.int32)
                    for c in range(4):
                        vals = src[l, pl.ds(c * _LANES, _LANES)]
                        plsc.store_scatter(dst, [base[c], lvec], vals)

                for dg in range(DG):
                    pltpu.async_copy(
                        tbuf.at[sl, pl.ds(dg * 8, 8), pl.ds(0, 128)],
                        out_hbm.at[s, dg, wid], osem.at[sl])

        for sl in range(2):
            drain_out(sl)

    o5 = emb(xT3, table)
    return o5.transpose(2, 4, 0, 1, 3).reshape(B, S, D)
